# baseline (speedup 1.0000x reference)
import jax
import jax.numpy as jnp
from jax import lax
from jax.experimental import pallas as pl
from jax.experimental.pallas import tpu as pltpu

E = 64
N = 32768
BT = 2048


def _gating_block(x_ref, o_ref):
    x = x_ref[...]
    m = jnp.max(x, axis=0, keepdims=True)
    eids = lax.broadcasted_iota(jnp.int32, (E, BT), 0)
    cand = jnp.where(x == m, eids, E)
    idx = jnp.min(cand, axis=0, keepdims=True)
    o_ref[...] = (eids == idx).astype(jnp.float32)


def kernel(snr_probs):
    k = pl.pallas_call(
        _gating_block,
        grid=(N // BT,),
        in_specs=[pl.BlockSpec((E, BT), lambda i: (0, i))],
        out_specs=pl.BlockSpec((E, BT), lambda i: (0, i)),
        out_shape=jax.ShapeDtypeStruct((E, N), jnp.float32),
        compiler_params=pltpu.CompilerParams(
            dimension_semantics=("arbitrary",),
        ),
    )
    x = pltpu.with_memory_space_constraint(snr_probs.T, pltpu.MemorySpace.HBM)
    return k(x).T


# speedup vs baseline: 1.0034x; 1.0034x over previous
import jax
import jax.numpy as jnp
from jax import lax
from jax.experimental import pallas as pl
from jax.experimental.pallas import tpu as pltpu

E = 64
N = 32768
BT = 2048


def _gating_block(x_ref, o_ref):
    x = x_ref[...]
    m = jnp.max(x, axis=0, keepdims=True)
    eids = lax.broadcasted_iota(jnp.int32, (E, BT), 0)
    cand = jnp.where(x == m, eids, E)
    idx = jnp.min(cand, axis=0, keepdims=True)
    o_ref[...] = (eids == idx).astype(jnp.float32)


def kernel(snr_probs):
    k = pl.pallas_call(
        _gating_block,
        grid=(N // BT,),
        in_specs=[pl.BlockSpec((E, BT), lambda i: (0, i))],
        out_specs=pl.BlockSpec((E, BT), lambda i: (0, i)),
        out_shape=jax.ShapeDtypeStruct((E, N), jnp.float32),
        compiler_params=pltpu.CompilerParams(
            dimension_semantics=("parallel",),
        ),
    )
    x = pltpu.with_memory_space_constraint(snr_probs.T, pltpu.MemorySpace.HBM)
    return k(x).T


# speedup vs baseline: 1.1071x; 1.1033x over previous
import jax
import jax.numpy as jnp
from jax import lax
from jax.experimental import pallas as pl
from jax.experimental.pallas import tpu as pltpu

E = 64
N = 32768
BT = 2048
NSTEPS = N // BT
NBUF = 2


def _gating_body(x_hbm, o_hbm, in_v, out_v, sem_in, sem_out):
    def in_copy(s, b):
        return pltpu.make_async_copy(
            x_hbm.at[:, pl.ds(s * BT, BT)], in_v.at[b], sem_in.at[b])

    def out_copy(s, b):
        return pltpu.make_async_copy(
            out_v.at[b], o_hbm.at[:, pl.ds(s * BT, BT)], sem_out.at[b])

    in_copy(0, 0).start()
    in_copy(1, 1).start()

    eids = lax.broadcasted_iota(jnp.int32, (E, BT), 0)
    for s in range(NSTEPS):
        b = s % NBUF
        in_copy(s, b).wait()
        if s >= NBUF:
            out_copy(s - NBUF, b).wait()
        x = in_v[b]
        m = jnp.max(x, axis=0, keepdims=True)
        cand = jnp.where(x == m, eids, E)
        idx = jnp.min(cand, axis=0, keepdims=True)
        out_v[b] = (eids == idx).astype(jnp.float32)
        out_copy(s, b).start()
        if s + NBUF < NSTEPS:
            in_copy(s + NBUF, b).start()

    out_copy(NSTEPS - 2, (NSTEPS - 2) % NBUF).wait()
    out_copy(NSTEPS - 1, (NSTEPS - 1) % NBUF).wait()


def kernel(snr_probs):
    k = pl.pallas_call(
        _gating_body,
        in_specs=[pl.BlockSpec(memory_space=pltpu.MemorySpace.HBM)],
        out_specs=pl.BlockSpec(memory_space=pltpu.MemorySpace.HBM),
        out_shape=jax.ShapeDtypeStruct((E, N), jnp.float32),
        scratch_shapes=[
            pltpu.VMEM((NBUF, E, BT), jnp.float32),
            pltpu.VMEM((NBUF, E, BT), jnp.float32),
            pltpu.SemaphoreType.DMA((NBUF,)),
            pltpu.SemaphoreType.DMA((NBUF,)),
        ],
    )
    x = pltpu.with_memory_space_constraint(snr_probs.T, pltpu.MemorySpace.HBM)
    return k(x).T


# speedup vs baseline: 1.6619x; 1.5012x over previous
import jax
import jax.numpy as jnp
from jax import lax
from jax.experimental import pallas as pl
from jax.experimental.pallas import tpu as pltpu

E = 64
N = 32768
BT = 2048
NSTEPS = N // BT
NBUF = 4


def _gating_body(x_hbm, o_hbm, in_v, out_v, sem_in, sem_out):
    def in_copy(s, b):
        return pltpu.make_async_copy(
            x_hbm.at[:, pl.ds(s * BT, BT)], in_v.at[b], sem_in.at[b])

    def out_copy(s, b):
        return pltpu.make_async_copy(
            out_v.at[b], o_hbm.at[:, pl.ds(s * BT, BT)], sem_out.at[b])

    for p in range(NBUF):
        in_copy(p, p).start()

    eids = lax.broadcasted_iota(jnp.int32, (E, BT), 0)
    for s in range(NSTEPS):
        b = s % NBUF
        in_copy(s, b).wait()
        if s >= NBUF:
            out_copy(s - NBUF, b).wait()
        x = in_v[b]
        m = jnp.max(x, axis=0, keepdims=True)
        cand = jnp.where(x == m, eids, E)
        idx = jnp.min(cand, axis=0, keepdims=True)
        out_v[b] = (eids == idx).astype(jnp.float32)
        out_copy(s, b).start()
        if s + NBUF < NSTEPS:
            in_copy(s + NBUF, b).start()

    for p in range(NSTEPS - NBUF, NSTEPS):
        out_copy(p, p % NBUF).wait()


def kernel(snr_probs):
    k = pl.pallas_call(
        _gating_body,
        in_specs=[pl.BlockSpec(memory_space=pltpu.MemorySpace.HBM)],
        out_specs=pl.BlockSpec(memory_space=pltpu.MemorySpace.HBM),
        out_shape=jax.ShapeDtypeStruct((E, N), jnp.float32),
        scratch_shapes=[
            pltpu.VMEM((NBUF, E, BT), jnp.float32),
            pltpu.VMEM((NBUF, E, BT), jnp.float32),
            pltpu.SemaphoreType.DMA((NBUF,)),
            pltpu.SemaphoreType.DMA((NBUF,)),
        ],
    )
    x = pltpu.with_memory_space_constraint(snr_probs.T, pltpu.MemorySpace.HBM)
    return k(x).T


# speedup vs baseline: 1.7462x; 1.0507x over previous
import jax
import jax.numpy as jnp
from jax import lax
from jax.experimental import pallas as pl
from jax.experimental.pallas import tpu as pltpu

E = 64
N = 32768
BT = 1024
NSTEPS = N // BT
NBUF = 8


def _gating_body(x_hbm, o_hbm, in_v, out_v, sem_in, sem_out):
    def in_copy(s, b):
        return pltpu.make_async_copy(
            x_hbm.at[:, pl.ds(s * BT, BT)], in_v.at[b], sem_in.at[b])

    def out_copy(s, b):
        return pltpu.make_async_copy(
            out_v.at[b], o_hbm.at[:, pl.ds(s * BT, BT)], sem_out.at[b])

    for p in range(NBUF):
        in_copy(p, p).start()

    eids = lax.broadcasted_iota(jnp.int32, (E, BT), 0)
    for s in range(NSTEPS):
        b = s % NBUF
        in_copy(s, b).wait()
        if s >= NBUF:
            out_copy(s - NBUF, b).wait()
        x = in_v[b]
        m = jnp.max(x, axis=0, keepdims=True)
        cand = jnp.where(x == m, eids, E)
        idx = jnp.min(cand, axis=0, keepdims=True)
        out_v[b] = (eids == idx).astype(jnp.float32)
        out_copy(s, b).start()
        if s + NBUF < NSTEPS:
            in_copy(s + NBUF, b).start()

    for p in range(NSTEPS - NBUF, NSTEPS):
        out_copy(p, p % NBUF).wait()


def kernel(snr_probs):
    k = pl.pallas_call(
        _gating_body,
        in_specs=[pl.BlockSpec(memory_space=pltpu.MemorySpace.HBM)],
        out_specs=pl.BlockSpec(memory_space=pltpu.MemorySpace.HBM),
        out_shape=jax.ShapeDtypeStruct((E, N), jnp.float32),
        scratch_shapes=[
            pltpu.VMEM((NBUF, E, BT), jnp.float32),
            pltpu.VMEM((NBUF, E, BT), jnp.float32),
            pltpu.SemaphoreType.DMA((NBUF,)),
            pltpu.SemaphoreType.DMA((NBUF,)),
        ],
    )
    x = pltpu.with_memory_space_constraint(snr_probs.T, pltpu.MemorySpace.HBM)
    return k(x).T
